# baseline (speedup 1.0000x reference)
import jax
import jax.numpy as jnp
from jax.experimental import pallas as pl
from jax.experimental.pallas import tpu as pltpu

_B = 16
_N = 16384


def _body(cu_ref, flat_ref, w1_ref, b1_ref, w2_ref, b2_ref, w3_ref, b3_ref,
          w4_ref, b4_ref, w5t_ref, b5_ref, out_ref,
          w1s_ref, b1s_ref, w2s_ref, w3s_ref, h1_ref, h2_ref):
    w1s_ref[...] = jnp.zeros_like(w1s_ref)
    w1s_ref[0:3, 0:64] = w1_ref[...]
    b1s_ref[...] = (jax.lax.broadcasted_iota(jnp.int32, (1, 72), 1)
                    == 64).astype(jnp.float32)
    b1s_ref[0:1, 0:64] = b1_ref[...].reshape(1, 64)
    w2s_ref[...] = jnp.zeros_like(w2s_ref)
    w2s_ref[0:64, 0:128] = w2_ref[...]
    w2s_ref[64:65, 0:128] = b2_ref[...].reshape(1, 128)
    w2s_ref[64:65, 128:136] = (
        jax.lax.broadcasted_iota(jnp.int32, (1, 8), 1) == 0
    ).astype(jnp.float32)
    w3s_ref[...] = jnp.zeros_like(w3s_ref)
    w3s_ref[0:128, :] = w3_ref[...]
    w3s_ref[128:129, :] = b3_ref[...].reshape(1, 256)

    xt = flat_ref[...]
    h1_ref[...] = jnp.maximum(
        jax.lax.dot_general(xt, w1s_ref[0:3, :], (((0,), (0,)), ((), ())),
                            preferred_element_type=jnp.float32)
        + b1s_ref[...], 0.0)
    h2_ref[...] = jnp.maximum(
        jnp.dot(h1_ref[...], w2s_ref[...],
                preferred_element_type=jnp.float32), 0.0)
    h = jnp.maximum(
        jnp.dot(h2_ref[...], w3s_ref[...],
                preferred_element_type=jnp.float32), 0.0)

    cut = jnp.transpose(cu_ref[...].reshape(1, 17))
    lo = cut[:16]
    hi = cut[1:17]
    invc = 1.0 / jnp.maximum(hi - lo, 1).astype(jnp.float32)
    row = jax.lax.broadcasted_iota(jnp.int32, (_B, _N), 1)
    onehot = ((row >= lo) & (row < hi)).astype(jnp.float32) * invc
    pooled = jnp.dot(onehot, h, preferred_element_type=jnp.float32)

    z = jnp.maximum(
        jnp.dot(pooled, w4_ref[...], preferred_element_type=jnp.float32)
        + b4_ref[...], 0.0)
    o = jax.lax.dot_general(
        z, w5t_ref[...], (((1,), (1,)), ((), ())),
        preferred_element_type=jnp.float32) + b5_ref[...]
    out_ref[...] = jax.nn.sigmoid(o)


def kernel(flat, cu_seqlens, W1, b1, W2, b2, W3, b3, W4, b4, W5, b5):
    full = lambda shape: pl.BlockSpec(shape, lambda: tuple(0 for _ in shape))
    in_specs = [
        full((17,)),
        full((3, _N)),
        full(W1.shape), full((64,)),
        full(W2.shape), full((128,)),
        full(W3.shape), full((256,)),
        full(W4.shape), full((512,)),
        full((2500, 512)), full((2500,)),
    ]
    return pl.pallas_call(
        _body,
        in_specs=in_specs,
        out_specs=full((_B, 2500)),
        out_shape=jax.ShapeDtypeStruct((_B, 2500), jnp.float32),
        scratch_shapes=[
            pltpu.VMEM((8, 72), jnp.float32),
            pltpu.VMEM((1, 72), jnp.float32),
            pltpu.VMEM((72, 136), jnp.float32),
            pltpu.VMEM((136, 256), jnp.float32),
            pltpu.VMEM((_N, 72), jnp.float32),
            pltpu.VMEM((_N, 136), jnp.float32),
        ],
    )(cu_seqlens, flat.T, W1, b1, W2, b2, W3, b3, W4, b4, W5.T, b5)


# speedup vs baseline: 1.1372x; 1.1372x over previous
import jax
import jax.numpy as jnp
from jax.experimental import pallas as pl
from jax.experimental.pallas import tpu as pltpu

_B = 16
_N = 16384


def _body(cu_ref, flat_ref, w1_ref, b1_ref, w2_ref, b2_ref, w3_ref, b3_ref,
          w4_ref, b4_ref, w5t_ref, b5_ref, out_ref,
          w1s_ref, b1s_ref, w2s_ref, w3s_ref, h1_ref, h2_ref,
          w5v_ref, w5_sem):
    w5_copy = pltpu.make_async_copy(w5t_ref, w5v_ref, w5_sem)
    w5_copy.start()

    w1s_ref[...] = jnp.zeros_like(w1s_ref)
    w1s_ref[0:3, 0:64] = w1_ref[...]
    b1s_ref[...] = (jax.lax.broadcasted_iota(jnp.int32, (1, 72), 1)
                    == 64).astype(jnp.float32)
    b1s_ref[0:1, 0:64] = b1_ref[...].reshape(1, 64)
    w2s_ref[...] = jnp.zeros_like(w2s_ref)
    w2s_ref[0:64, 0:128] = w2_ref[...]
    w2s_ref[64:65, 0:128] = b2_ref[...].reshape(1, 128)
    w2s_ref[64:65, 128:136] = (
        jax.lax.broadcasted_iota(jnp.int32, (1, 8), 1) == 0
    ).astype(jnp.float32)
    w3s_ref[...] = jnp.zeros_like(w3s_ref)
    w3s_ref[0:128, :] = w3_ref[...]
    w3s_ref[128:129, :] = b3_ref[...].reshape(1, 256)

    xt = flat_ref[...]
    h1_ref[...] = jnp.maximum(
        jax.lax.dot_general(xt, w1s_ref[0:3, :], (((0,), (0,)), ((), ())),
                            preferred_element_type=jnp.float32)
        + b1s_ref[...], 0.0)
    h2_ref[...] = jnp.maximum(
        jnp.dot(h1_ref[...], w2s_ref[...],
                preferred_element_type=jnp.float32), 0.0)
    h = jnp.maximum(
        jnp.dot(h2_ref[...], w3s_ref[...],
                preferred_element_type=jnp.float32), 0.0)

    cut = jnp.transpose(cu_ref[...].reshape(1, 17))
    lo = cut[:16]
    hi = cut[1:17]
    invc = 1.0 / jnp.maximum(hi - lo, 1).astype(jnp.float32)
    row = jax.lax.broadcasted_iota(jnp.int32, (_B, _N), 1)
    onehot = ((row >= lo) & (row < hi)).astype(jnp.float32) * invc
    pooled = jnp.dot(onehot, h, preferred_element_type=jnp.float32)

    z = jnp.maximum(
        jnp.dot(pooled, w4_ref[...], preferred_element_type=jnp.float32)
        + b4_ref[...], 0.0)
    w5_copy.wait()
    o = jax.lax.dot_general(
        z, w5v_ref[...], (((1,), (1,)), ((), ())),
        preferred_element_type=jnp.float32) + b5_ref[...]
    out_ref[...] = jax.nn.sigmoid(o)


def kernel(flat, cu_seqlens, W1, b1, W2, b2, W3, b3, W4, b4, W5, b5):
    full = lambda shape: pl.BlockSpec(shape, lambda: tuple(0 for _ in shape))
    in_specs = [
        full((17,)),
        full((3, _N)),
        full(W1.shape), full((64,)),
        full(W2.shape), full((128,)),
        full(W3.shape), full((256,)),
        full(W4.shape), full((512,)),
        pl.BlockSpec(memory_space=pltpu.MemorySpace.HBM),
        full((2500,)),
    ]
    return pl.pallas_call(
        _body,
        in_specs=in_specs,
        out_specs=full((_B, 2500)),
        out_shape=jax.ShapeDtypeStruct((_B, 2500), jnp.float32),
        scratch_shapes=[
            pltpu.VMEM((8, 72), jnp.float32),
            pltpu.VMEM((1, 72), jnp.float32),
            pltpu.VMEM((72, 136), jnp.float32),
            pltpu.VMEM((136, 256), jnp.float32),
            pltpu.VMEM((_N, 72), jnp.float32),
            pltpu.VMEM((_N, 136), jnp.float32),
            pltpu.VMEM((2500, 512), jnp.float32),
            pltpu.SemaphoreType.DMA,
        ],
    )(cu_seqlens, flat.T, W1, b1, W2, b2, W3, b3, W4, b4, W5.T, b5)
